# baseline (speedup 1.0000x reference)
import functools

import jax
import jax.numpy as jnp
from jax import lax
from jax.experimental import pallas as pl
from jax.experimental.pallas import tpu as pltpu
from jax.experimental.pallas import tpu_sc as plsc

_NUM_CORES = 2
_NUM_SUBCORES = 16
_NW = _NUM_CORES * _NUM_SUBCORES

_CHUNK = 32
_NBUF = 8
_F = 4


def _make_gather(b: int, n: int, d: int, band_dtype, rho_dtype):
    total_rows = b * n
    rows_per_worker = total_rows // _NW
    n_chunks = rows_per_worker // _CHUNK
    workers_per_batch = n // rows_per_worker
    meta_per_worker = n // _NW
    assert n_chunks % _NBUF == 0 and workers_per_batch * rows_per_worker == n
    mesh = plsc.VectorSubcoreMesh(
        core_axis_name="c",
        subcore_axis_name="s",
        num_cores=_NUM_CORES,
        num_subcores=_NUM_SUBCORES,
    )

    @functools.partial(
        pl.kernel,
        mesh=mesh,
        out_type=(
            jax.ShapeDtypeStruct((b, n, d), jnp.float32),
            jax.ShapeDtypeStruct((n,), band_dtype),
            jax.ShapeDtypeStruct((n,), rho_dtype),
        ),
        scratch_types=[
            pltpu.VMEM((rows_per_worker,), jnp.int32),
            pltpu.VMEM((meta_per_worker,), band_dtype),
            pltpu.VMEM((meta_per_worker,), rho_dtype),
            *[pltpu.VMEM((_CHUNK, d), jnp.float32) for _ in range(_NBUF)],
            *[pltpu.SemaphoreType.DMA for _ in range(2 * _NBUF)],
        ],
    )
    def gather(
        x_hbm,
        fo_hbm,
        band_hbm,
        rho_hbm,
        out_hbm,
        band_out,
        rho_out,
        idx_v,
        band_v,
        rho_v,
        *rest,
    ):
        bufs = rest[:_NBUF]
        gsem = rest[_NBUF : 2 * _NBUF]
        ssem = rest[2 * _NBUF :]
        wid = lax.axis_index("s") * _NUM_CORES + lax.axis_index("c")
        batch = wid // workers_per_batch
        row0 = (wid % workers_per_batch) * rows_per_worker

        def fire_gather(c, j):
            pltpu.async_copy(
                x_hbm.at[batch].at[idx_v.at[pl.ds(c * _CHUNK, _CHUNK)]],
                bufs[j],
                gsem[j],
            )

        def wait_gather(c, j):
            pltpu.make_async_copy(
                x_hbm.at[batch].at[idx_v.at[pl.ds(c * _CHUNK, _CHUNK)]],
                bufs[j],
                gsem[j],
            ).wait()

        def out_slice(c):
            return out_hbm.at[batch].at[pl.ds(row0 + c * _CHUNK, _CHUNK)]

        def fire_store(c, j):
            pltpu.async_copy(bufs[j], out_slice(c), ssem[j])

        def wait_store(c, j):
            pltpu.make_async_copy(bufs[j], out_slice(c), ssem[j]).wait()

        pltpu.sync_copy(fo_hbm.at[pl.ds(row0, rows_per_worker)], idx_v)
        for c0 in range(_F):
            fire_gather(c0, c0 % _NBUF)

        meta0 = wid * meta_per_worker
        pltpu.sync_copy(band_hbm.at[pl.ds(meta0, meta_per_worker)], band_v)
        pltpu.sync_copy(band_v, band_out.at[pl.ds(meta0, meta_per_worker)])
        pltpu.sync_copy(rho_hbm.at[pl.ds(meta0, meta_per_worker)], rho_v)
        pltpu.sync_copy(rho_v, rho_out.at[pl.ds(meta0, meta_per_worker)])

        _W = _NBUF - _F

        @pl.loop(0, n_chunks, step=_NBUF)
        def _(g):
            for j in range(_NBUF):
                c = g + j
                jj = (j + _F) % _NBUF

                @pl.when(c >= _W)
                def _():
                    wait_store(c - _W, jj)

                @pl.when(c + _F < n_chunks)
                def _():
                    fire_gather(c + _F, jj)

                wait_gather(c, j)
                fire_store(c, j)

        for c0 in range(n_chunks - _NBUF + _F, n_chunks):
            wait_store(c0, c0 % _NBUF)

    return gather


def kernel(x, forward_order, band_indices, rho_bar):
    b, n, d = x.shape
    fo = forward_order.astype(jnp.int32)
    out, band_out, rho_out = _make_gather(
        b, n, d, band_indices.dtype, rho_bar.dtype
    )(x, fo, band_indices, rho_bar)
    return (out, band_out, rho_out)


# speedup vs baseline: 1.0081x; 1.0081x over previous
import functools

import jax
import jax.numpy as jnp
from jax import lax
from jax.experimental import pallas as pl
from jax.experimental.pallas import tpu as pltpu
from jax.experimental.pallas import tpu_sc as plsc

_NUM_CORES = 2
_NUM_SUBCORES = 16
_NW = _NUM_CORES * _NUM_SUBCORES

_CHUNK = 32
_NBUF = 8
_F = 3


def _make_gather(b: int, n: int, d: int, band_dtype, rho_dtype):
    total_rows = b * n
    rows_per_worker = total_rows // _NW
    n_chunks = rows_per_worker // _CHUNK
    workers_per_batch = n // rows_per_worker
    meta_per_worker = n // _NW
    assert n_chunks % _NBUF == 0 and workers_per_batch * rows_per_worker == n
    mesh = plsc.VectorSubcoreMesh(
        core_axis_name="c",
        subcore_axis_name="s",
        num_cores=_NUM_CORES,
        num_subcores=_NUM_SUBCORES,
    )

    @functools.partial(
        pl.kernel,
        mesh=mesh,
        out_type=(
            jax.ShapeDtypeStruct((b, n, d), jnp.float32),
            jax.ShapeDtypeStruct((n,), band_dtype),
            jax.ShapeDtypeStruct((n,), rho_dtype),
        ),
        scratch_types=[
            pltpu.VMEM((rows_per_worker,), jnp.int32),
            pltpu.VMEM((meta_per_worker,), band_dtype),
            pltpu.VMEM((meta_per_worker,), rho_dtype),
            *[pltpu.VMEM((_CHUNK, d), jnp.float32) for _ in range(_NBUF)],
            *[pltpu.SemaphoreType.DMA for _ in range(2 * _NBUF)],
        ],
    )
    def gather(
        x_hbm,
        fo_hbm,
        band_hbm,
        rho_hbm,
        out_hbm,
        band_out,
        rho_out,
        idx_v,
        band_v,
        rho_v,
        *rest,
    ):
        bufs = rest[:_NBUF]
        gsem = rest[_NBUF : 2 * _NBUF]
        ssem = rest[2 * _NBUF :]
        wid = lax.axis_index("s") * _NUM_CORES + lax.axis_index("c")
        batch = wid // workers_per_batch
        row0 = (wid % workers_per_batch) * rows_per_worker

        def fire_gather(c, j):
            pltpu.async_copy(
                x_hbm.at[batch].at[idx_v.at[pl.ds(c * _CHUNK, _CHUNK)]],
                bufs[j],
                gsem[j],
            )

        def wait_gather(c, j):
            pltpu.make_async_copy(
                x_hbm.at[batch].at[idx_v.at[pl.ds(c * _CHUNK, _CHUNK)]],
                bufs[j],
                gsem[j],
            ).wait()

        def out_slice(c):
            return out_hbm.at[batch].at[pl.ds(row0 + c * _CHUNK, _CHUNK)]

        def fire_store(c, j):
            pltpu.async_copy(bufs[j], out_slice(c), ssem[j])

        def wait_store(c, j):
            pltpu.make_async_copy(bufs[j], out_slice(c), ssem[j]).wait()

        pltpu.sync_copy(fo_hbm.at[pl.ds(row0, rows_per_worker)], idx_v)
        for c0 in range(_F):
            fire_gather(c0, c0 % _NBUF)

        meta0 = wid * meta_per_worker
        pltpu.sync_copy(band_hbm.at[pl.ds(meta0, meta_per_worker)], band_v)
        pltpu.sync_copy(band_v, band_out.at[pl.ds(meta0, meta_per_worker)])
        pltpu.sync_copy(rho_hbm.at[pl.ds(meta0, meta_per_worker)], rho_v)
        pltpu.sync_copy(rho_v, rho_out.at[pl.ds(meta0, meta_per_worker)])

        _W = _NBUF - _F

        @pl.loop(0, n_chunks, step=_NBUF)
        def _(g):
            for j in range(_NBUF):
                c = g + j
                jj = (j + _F) % _NBUF

                @pl.when(c >= _W)
                def _():
                    wait_store(c - _W, jj)

                @pl.when(c + _F < n_chunks)
                def _():
                    fire_gather(c + _F, jj)

                wait_gather(c, j)
                fire_store(c, j)

        for c0 in range(n_chunks - _NBUF + _F, n_chunks):
            wait_store(c0, c0 % _NBUF)

    return gather


def kernel(x, forward_order, band_indices, rho_bar):
    b, n, d = x.shape
    fo = forward_order.astype(jnp.int32)
    out, band_out, rho_out = _make_gather(
        b, n, d, band_indices.dtype, rho_bar.dtype
    )(x, fo, band_indices, rho_bar)
    return (out, band_out, rho_out)
